# baseline (speedup 1.0000x reference)
import functools

import jax
import jax.numpy as jnp
from jax import lax
from jax.experimental import pallas as pl
from jax.experimental.pallas import tpu as pltpu
from jax.experimental.pallas import tpu_sc as plsc

N = 10000
E = 320000
NC = 2
NS = 16
NW = NC * NS
NROW = N + 8

PCHUNKS = {128: (80, 125), 64: (128, 79), 48: (128, 79)}
EPAD = NW * 79 * 128

DEG_CHUNK = 80
DEG_NCHUNK = 125
DEG_D = 16
DEG_FIRE = 5

ROWS_PT = 624
ROWS_TAIL = NROW - NS * ROWS_PT


def _make_propagate(d):
  PCHUNK, PNCHUNK = PCHUNKS[d]
  mesh = plsc.VectorSubcoreMesh(core_axis_name="c", subcore_axis_name="s")

  @functools.partial(
      pl.kernel,
      out_type=jax.ShapeDtypeStruct((NC, NROW, d), jnp.float32),
      mesh=mesh,
      scratch_types=[
          pltpu.VMEM((PNCHUNK, PCHUNK), jnp.int32),
          pltpu.VMEM((PNCHUNK, PCHUNK), jnp.int32),
          pltpu.VMEM((PCHUNK, d), jnp.float32),
          pltpu.VMEM((PCHUNK, d), jnp.float32),
          pltpu.VMEM_SHARED((NROW, d), jnp.float32),
          pltpu.SemaphoreType.DMA,
          pltpu.SemaphoreType.DMA,
      ],
      compiler_params=pltpu.CompilerParams(use_tc_tiling_on_sc=False),
  )
  def propagate(g_hbm, src_hbm, dst_hbm, zeros_hbm, out_hbm,
                src_v, dst_v, rows0, rows1, acc, sem0, sem1):
    cid = lax.axis_index("c")
    sid = lax.axis_index("s")
    wid = cid * NS + sid
    pltpu.sync_copy(src_hbm.at[wid], src_v)
    pltpu.sync_copy(dst_hbm.at[wid], dst_v)
    row0 = sid * ROWS_PT
    pltpu.sync_copy(zeros_hbm.at[pl.ds(row0, ROWS_PT)],
                    acc.at[pl.ds(row0, ROWS_PT)])

    @pl.when(sid == 0)
    def _zero_tail():
      pltpu.sync_copy(zeros_hbm.at[pl.ds(NS * ROWS_PT, ROWS_TAIL)],
                      acc.at[pl.ds(NS * ROWS_PT, ROWS_TAIL)])

    plsc.subcore_barrier()

    pltpu.async_copy(g_hbm.at[src_v.at[0]], rows0, sem0)

    @pl.loop(0, PNCHUNK - 1, step=2)
    def _pair(j):
      pltpu.async_copy(g_hbm.at[src_v.at[j + 1]], rows1, sem1)
      pltpu.make_async_copy(g_hbm.at[src_v.at[j]], rows0, sem0).wait()
      pltpu.sync_copy(rows0, acc.at[dst_v.at[j]], add=True)
      pltpu.async_copy(g_hbm.at[src_v.at[j + 2]], rows0, sem0)
      pltpu.make_async_copy(g_hbm.at[src_v.at[j + 1]], rows1, sem1).wait()
      pltpu.sync_copy(rows1, acc.at[dst_v.at[j + 1]], add=True)

    pltpu.make_async_copy(g_hbm.at[src_v.at[PNCHUNK - 1]], rows0, sem0).wait()
    pltpu.sync_copy(rows0, acc.at[dst_v.at[PNCHUNK - 1]], add=True)

    plsc.subcore_barrier()
    pltpu.sync_copy(acc.at[pl.ds(row0, ROWS_PT)],
                    out_hbm.at[cid, pl.ds(row0, ROWS_PT)])

    @pl.when(sid == 0)
    def _copy_tail():
      pltpu.sync_copy(acc.at[pl.ds(NS * ROWS_PT, ROWS_TAIL)],
                      out_hbm.at[cid, pl.ds(NS * ROWS_PT, ROWS_TAIL)])

  return propagate


_PROPAGATE = {d: _make_propagate(d) for d in (128, 64, 48)}


def _make_degree():
  mesh = plsc.VectorSubcoreMesh(core_axis_name="c", subcore_axis_name="s")

  @functools.partial(
      pl.kernel,
      out_type=jax.ShapeDtypeStruct((NC, NROW, DEG_D), jnp.float32),
      mesh=mesh,
      scratch_types=[
          pltpu.VMEM((DEG_NCHUNK, DEG_CHUNK), jnp.int32),
          pltpu.VMEM((DEG_CHUNK, DEG_D), jnp.float32),
          pltpu.VMEM_SHARED((NROW, DEG_D), jnp.float32),
          pltpu.SemaphoreType.DMA,
      ],
      compiler_params=pltpu.CompilerParams(use_tc_tiling_on_sc=False),
  )
  def degree(ones_hbm, dst_hbm, zeros_hbm, out_hbm, dst_v, ones_v, acc, sem):
    cid = lax.axis_index("c")
    sid = lax.axis_index("s")
    wid = cid * NS + sid
    pltpu.sync_copy(dst_hbm.at[wid], dst_v)
    pltpu.sync_copy(ones_hbm, ones_v)
    row0 = sid * ROWS_PT
    pltpu.sync_copy(zeros_hbm.at[pl.ds(row0, ROWS_PT)],
                    acc.at[pl.ds(row0, ROWS_PT)])

    @pl.when(sid == 0)
    def _zero_tail():
      pltpu.sync_copy(zeros_hbm.at[pl.ds(NS * ROWS_PT, ROWS_TAIL)],
                      acc.at[pl.ds(NS * ROWS_PT, ROWS_TAIL)])

    plsc.subcore_barrier()

    @pl.loop(0, DEG_NCHUNK, step=DEG_FIRE)
    def _group(j):
      for k in range(DEG_FIRE):
        pltpu.async_copy(ones_v, acc.at[dst_v.at[j + k]], sem, add=True)
      for k in range(DEG_FIRE):
        pltpu.make_async_copy(ones_v, acc.at[dst_v.at[j + k]], sem).wait()

    plsc.subcore_barrier()
    pltpu.sync_copy(acc.at[pl.ds(row0, ROWS_PT)],
                    out_hbm.at[cid, pl.ds(row0, ROWS_PT)])

    @pl.when(sid == 0)
    def _copy_tail():
      pltpu.sync_copy(acc.at[pl.ds(NS * ROWS_PT, ROWS_TAIL)],
                      out_hbm.at[cid, pl.ds(NS * ROWS_PT, ROWS_TAIL)])

  return degree


_DEGREE = _make_degree()


def _tc_call(body, out_shape, *args):
  return pl.pallas_call(body, out_shape=out_shape)(*args)


def _deg_scale_body(dp_ref, x_ref, dinv_ref, g_ref):
  deg = dp_ref[0, :, 0:1] + dp_ref[1, :, 0:1] + 2.0
  dinv = lax.rsqrt(deg)
  dinv_ref[...] = dinv
  g_ref[0:N, :] = x_ref[...] * dinv[0:N]


def _layer_relu_mm_body(p_ref, g_ref, dinv_ref, w_ref, b_ref, w2_ref, out_ref):
  s = dinv_ref[...] * (p_ref[0] + p_ref[1] + 2.0 * g_ref[...])
  z = jax.nn.relu(
      jnp.dot(s, w_ref[...], preferred_element_type=jnp.float32) + b_ref[...])
  out_ref[...] = dinv_ref[...] * jnp.dot(
      z, w2_ref[...], preferred_element_type=jnp.float32)


def _layer_relu_ew_mm_body(p_ref, g_ref, dinv_ref, b_ref, w2_ref, out_ref):
  z = jax.nn.relu(
      dinv_ref[...] * (p_ref[0] + p_ref[1] + 2.0 * g_ref[...]) + b_ref[...])
  out_ref[...] = dinv_ref[...] * jnp.dot(
      z, w2_ref[...], preferred_element_type=jnp.float32)


def _layer_lin_mm_body(p_ref, g_ref, dinv_ref, b_ref, w2_ref, out_ref):
  z = dinv_ref[...] * (p_ref[0] + p_ref[1] + 2.0 * g_ref[...]) + b_ref[...]
  out_ref[...] = dinv_ref[...] * jnp.dot(
      z, w2_ref[...], preferred_element_type=jnp.float32)


def _final_body(p_ref, g_ref, dinv_ref, b_ref, out_ref):
  out_ref[...] = (dinv_ref[...] * (p_ref[0] + p_ref[1] + 2.0 * g_ref[...])
                  + b_ref[...])


def kernel(x, edge_index, W1, b1, W2, b2, W3, b3, W4, b4):
  i32 = jnp.int32
  f32 = jnp.float32
  pad = jnp.full((EPAD - E,), N, i32)
  src80 = edge_index[0].reshape(NW, 125, 80)
  dst80 = edge_index[1].reshape(NW, 125, 80)
  src128 = jnp.concatenate([edge_index[0], pad]).reshape(NW, 79, 128)
  dst128 = jnp.concatenate([edge_index[1], pad]).reshape(NW, 79, 128)
  dst_deg = edge_index[1].reshape(NW, DEG_NCHUNK, DEG_CHUNK)

  zeros_flat = jnp.zeros((NROW * 128,), f32)
  zeros = {d: zeros_flat[:NROW * d].reshape(NROW, d) for d in (128, 64, 48)}
  zeros16 = zeros_flat[:NROW * DEG_D].reshape(NROW, DEG_D)
  ones16 = jnp.ones((DEG_CHUNK, DEG_D), f32)
  W4p = jnp.pad(W4, ((0, 0), (0, 8)))
  b4p = jnp.pad(b4, (0, 8))

  dp = _DEGREE(ones16, dst_deg, zeros16)
  dinv, g0 = _tc_call(
      _deg_scale_body,
      (jax.ShapeDtypeStruct((NROW, 1), f32),
       jax.ShapeDtypeStruct((NROW, 128), f32)),
      dp, x)

  s0 = _PROPAGATE[128](g0, src80, dst80, zeros[128])
  g1 = _tc_call(_layer_relu_mm_body, jax.ShapeDtypeStruct((NROW, 128), f32),
                s0, g0, dinv, W1, b1.reshape(1, -1), W2)

  s1 = _PROPAGATE[128](g1, src80, dst80, zeros[128])
  g2 = _tc_call(_layer_relu_ew_mm_body, jax.ShapeDtypeStruct((NROW, 64), f32),
                s1, g1, dinv, b2.reshape(1, -1), W3)

  s2 = _PROPAGATE[64](g2, src128, dst128, zeros[64])
  g3 = _tc_call(_layer_lin_mm_body, jax.ShapeDtypeStruct((NROW, 48), f32),
                s2, g2, dinv, b3.reshape(1, -1), W4p)

  s3 = _PROPAGATE[48](g3, src128, dst128, zeros[48])
  out = _tc_call(_final_body, jax.ShapeDtypeStruct((NROW, 48), f32),
                 s3, g3, dinv, b4p.reshape(1, -1))
  return out[:N, :40]


# speedup vs baseline: 1.0022x; 1.0022x over previous
import functools

import jax
import jax.numpy as jnp
from jax import lax
from jax.experimental import pallas as pl
from jax.experimental.pallas import tpu as pltpu
from jax.experimental.pallas import tpu_sc as plsc

N = 10000
E = 320000
NC = 2
NS = 16
NW = NC * NS
NROW = N + 8

PCHUNKS = {128: (80, 125), 64: (128, 79), 48: (128, 79)}
EPAD = NW * 79 * 128

DEG_CHUNK = 80
DEG_NCHUNK = 125
DEG_D = 16
DEG_FIRE = 5

ROWS_PT = 624
ROWS_TAIL = NROW - NS * ROWS_PT


def _make_propagate(d):
  PCHUNK, PNCHUNK = PCHUNKS[d]
  mesh = plsc.VectorSubcoreMesh(core_axis_name="c", subcore_axis_name="s")

  @functools.partial(
      pl.kernel,
      out_type=jax.ShapeDtypeStruct((NC, NROW, d), jnp.float32),
      mesh=mesh,
      scratch_types=[
          pltpu.VMEM((PNCHUNK, PCHUNK), jnp.int32),
          pltpu.VMEM((PNCHUNK, PCHUNK), jnp.int32),
          pltpu.VMEM((PCHUNK, d), jnp.float32),
          pltpu.VMEM((PCHUNK, d), jnp.float32),
          pltpu.VMEM_SHARED((NROW, d), jnp.float32),
          pltpu.SemaphoreType.DMA,
          pltpu.SemaphoreType.DMA,
      ],
      compiler_params=pltpu.CompilerParams(use_tc_tiling_on_sc=False),
  )
  def propagate(g_hbm, src_hbm, dst_hbm, zeros_hbm, out_hbm,
                src_v, dst_v, rows0, rows1, acc, sem0, sem1):
    cid = lax.axis_index("c")
    sid = lax.axis_index("s")
    wid = cid * NS + sid
    pltpu.sync_copy(src_hbm.at[wid], src_v)
    pltpu.sync_copy(dst_hbm.at[wid], dst_v)
    row0 = sid * ROWS_PT
    pltpu.sync_copy(zeros_hbm.at[pl.ds(row0, ROWS_PT)],
                    acc.at[pl.ds(row0, ROWS_PT)])

    @pl.when(sid == 0)
    def _zero_tail():
      pltpu.sync_copy(zeros_hbm.at[pl.ds(NS * ROWS_PT, ROWS_TAIL)],
                      acc.at[pl.ds(NS * ROWS_PT, ROWS_TAIL)])

    plsc.subcore_barrier()

    pltpu.async_copy(g_hbm.at[src_v.at[0]], rows0, sem0)

    @pl.loop(0, PNCHUNK - 1, step=2)
    def _pair(j):
      pltpu.async_copy(g_hbm.at[src_v.at[j + 1]], rows1, sem1)
      pltpu.make_async_copy(g_hbm.at[src_v.at[j]], rows0, sem0).wait()
      pltpu.sync_copy(rows0, acc.at[dst_v.at[j]], add=True)
      pltpu.async_copy(g_hbm.at[src_v.at[j + 2]], rows0, sem0)
      pltpu.make_async_copy(g_hbm.at[src_v.at[j + 1]], rows1, sem1).wait()
      pltpu.sync_copy(rows1, acc.at[dst_v.at[j + 1]], add=True)

    pltpu.make_async_copy(g_hbm.at[src_v.at[PNCHUNK - 1]], rows0, sem0).wait()
    pltpu.sync_copy(rows0, acc.at[dst_v.at[PNCHUNK - 1]], add=True)

    plsc.subcore_barrier()
    pltpu.sync_copy(acc.at[pl.ds(row0, ROWS_PT)],
                    out_hbm.at[cid, pl.ds(row0, ROWS_PT)])

    @pl.when(sid == 0)
    def _copy_tail():
      pltpu.sync_copy(acc.at[pl.ds(NS * ROWS_PT, ROWS_TAIL)],
                      out_hbm.at[cid, pl.ds(NS * ROWS_PT, ROWS_TAIL)])

  return propagate


_PROPAGATE = {d: _make_propagate(d) for d in (128, 64, 48)}


def _make_degree():
  mesh = plsc.VectorSubcoreMesh(core_axis_name="c", subcore_axis_name="s")

  @functools.partial(
      pl.kernel,
      out_type=jax.ShapeDtypeStruct((NC, NROW, DEG_D), jnp.float32),
      mesh=mesh,
      scratch_types=[
          pltpu.VMEM((DEG_NCHUNK, DEG_CHUNK), jnp.int32),
          pltpu.VMEM((DEG_CHUNK, DEG_D), jnp.float32),
          pltpu.VMEM_SHARED((NROW, DEG_D), jnp.float32),
          pltpu.SemaphoreType.DMA,
      ],
      compiler_params=pltpu.CompilerParams(use_tc_tiling_on_sc=False),
  )
  def degree(ones_hbm, dst_hbm, zeros_hbm, out_hbm, dst_v, ones_v, acc, sem):
    cid = lax.axis_index("c")
    sid = lax.axis_index("s")
    wid = cid * NS + sid
    pltpu.sync_copy(dst_hbm.at[wid], dst_v)
    pltpu.sync_copy(ones_hbm, ones_v)
    row0 = sid * ROWS_PT
    pltpu.sync_copy(zeros_hbm.at[pl.ds(row0, ROWS_PT)],
                    acc.at[pl.ds(row0, ROWS_PT)])

    @pl.when(sid == 0)
    def _zero_tail():
      pltpu.sync_copy(zeros_hbm.at[pl.ds(NS * ROWS_PT, ROWS_TAIL)],
                      acc.at[pl.ds(NS * ROWS_PT, ROWS_TAIL)])

    plsc.subcore_barrier()

    @pl.loop(0, DEG_NCHUNK, step=DEG_FIRE)
    def _group(j):
      for k in range(DEG_FIRE):
        pltpu.async_copy(ones_v, acc.at[dst_v.at[j + k]], sem, add=True)
      for k in range(DEG_FIRE):
        pltpu.make_async_copy(ones_v, acc.at[dst_v.at[j + k]], sem).wait()

    plsc.subcore_barrier()
    pltpu.sync_copy(acc.at[pl.ds(row0, ROWS_PT)],
                    out_hbm.at[cid, pl.ds(row0, ROWS_PT)])

    @pl.when(sid == 0)
    def _copy_tail():
      pltpu.sync_copy(acc.at[pl.ds(NS * ROWS_PT, ROWS_TAIL)],
                      out_hbm.at[cid, pl.ds(NS * ROWS_PT, ROWS_TAIL)])

  return degree


_DEGREE = _make_degree()


def _tc_call(body, out_shape, *args):
  return pl.pallas_call(body, out_shape=out_shape)(*args)


def _deg_scale_body(dp_ref, x_ref, dinv_ref, g_ref):
  deg = dp_ref[0, :, 0:1] + dp_ref[1, :, 0:1] + 2.0
  dinv = lax.rsqrt(deg)
  dinv_ref[...] = dinv
  g_ref[0:N, :] = x_ref[...] * dinv[0:N]


def _layer_relu_mm_body(p_ref, g_ref, dinv_ref, w_ref, b_ref, w2_ref, out_ref):
  s = dinv_ref[...] * (p_ref[0] + p_ref[1] + 2.0 * g_ref[...])
  z = jax.nn.relu(
      jnp.dot(s, w_ref[...], preferred_element_type=jnp.float32) + b_ref[...])
  out_ref[...] = dinv_ref[...] * jnp.dot(
      z, w2_ref[...], preferred_element_type=jnp.float32)


def _layer_relu_ew_mm_body(p_ref, g_ref, dinv_ref, b_ref, w2_ref, out_ref):
  z = jax.nn.relu(
      dinv_ref[...] * (p_ref[0] + p_ref[1] + 2.0 * g_ref[...]) + b_ref[...])
  out_ref[...] = dinv_ref[...] * jnp.dot(
      z, w2_ref[...], preferred_element_type=jnp.float32)
  out_ref[N:NROW, :] = jnp.zeros((NROW - N, w2_ref.shape[1]), jnp.float32)


def _layer_lin_mm_body(p_ref, g_ref, dinv_ref, b_ref, w2_ref, out_ref):
  z = dinv_ref[...] * (p_ref[0] + p_ref[1] + 2.0 * g_ref[...]) + b_ref[...]
  out_ref[...] = dinv_ref[...] * jnp.dot(
      z, w2_ref[...], preferred_element_type=jnp.float32)
  out_ref[N:NROW, :] = jnp.zeros((NROW - N, w2_ref.shape[1]), jnp.float32)


def _final_body(p_ref, g_ref, dinv_ref, b_ref, out_ref):
  out_ref[...] = (dinv_ref[...] * (p_ref[0] + p_ref[1] + 2.0 * g_ref[...])
                  + b_ref[...])


def kernel(x, edge_index, W1, b1, W2, b2, W3, b3, W4, b4):
  i32 = jnp.int32
  f32 = jnp.float32
  src_pad = jnp.full((EPAD - E,), N, i32)
  dst_pad = jnp.arange(EPAD - E, dtype=i32)
  src80 = edge_index[0].reshape(NW, 125, 80)
  dst80 = edge_index[1].reshape(NW, 125, 80)
  src128 = jnp.concatenate([edge_index[0], src_pad]).reshape(NW, 79, 128)
  dst128 = jnp.concatenate([edge_index[1], dst_pad]).reshape(NW, 79, 128)
  dst_deg = edge_index[1].reshape(NW, DEG_NCHUNK, DEG_CHUNK)

  zeros_flat = jnp.zeros((NROW * 128,), f32)
  zeros = {d: zeros_flat[:NROW * d].reshape(NROW, d) for d in (128, 64, 48)}
  zeros16 = zeros_flat[:NROW * DEG_D].reshape(NROW, DEG_D)
  ones16 = jnp.ones((DEG_CHUNK, DEG_D), f32)
  W4p = jnp.pad(W4, ((0, 0), (0, 8)))
  b4p = jnp.pad(b4, (0, 8))

  dp = _DEGREE(ones16, dst_deg, zeros16)
  dinv, g0 = _tc_call(
      _deg_scale_body,
      (jax.ShapeDtypeStruct((NROW, 1), f32),
       jax.ShapeDtypeStruct((NROW, 128), f32)),
      dp, x)

  s0 = _PROPAGATE[128](g0, src80, dst80, zeros[128])
  g1 = _tc_call(_layer_relu_mm_body, jax.ShapeDtypeStruct((NROW, 128), f32),
                s0, g0, dinv, W1, b1.reshape(1, -1), W2)

  s1 = _PROPAGATE[128](g1, src80, dst80, zeros[128])
  g2 = _tc_call(_layer_relu_ew_mm_body, jax.ShapeDtypeStruct((NROW, 64), f32),
                s1, g1, dinv, b2.reshape(1, -1), W3)

  s2 = _PROPAGATE[64](g2, src128, dst128, zeros[64])
  g3 = _tc_call(_layer_lin_mm_body, jax.ShapeDtypeStruct((NROW, 48), f32),
                s2, g2, dinv, b3.reshape(1, -1), W4p)

  s3 = _PROPAGATE[48](g3, src128, dst128, zeros[48])
  out = _tc_call(_final_body, jax.ShapeDtypeStruct((NROW, 48), f32),
                 s3, g3, dinv, b4p.reshape(1, -1))
  return out[:N, :40]


# speedup vs baseline: 1.2512x; 1.2484x over previous
import functools

import jax
import jax.numpy as jnp
from jax import lax
from jax.experimental import pallas as pl
from jax.experimental.pallas import tpu as pltpu
from jax.experimental.pallas import tpu_sc as plsc

N = 10000
E = 320000
NC = 2
NS = 16
NW = NC * NS
NROW = N + 128

PCHUNKS = {128: (80, 125), 64: (128, 79), 48: (128, 79)}
EPAD = NW * 79 * 128

DEG_CHUNK = 80
DEG_NCHUNK = 125
DEG_D = 16
DEG_FIRE = 5

ROWS_PT = 624
ROWS_TAIL = NROW - NS * ROWS_PT


def _make_propagate(d):
  PCHUNK, PNCHUNK = PCHUNKS[d]
  mesh = plsc.VectorSubcoreMesh(core_axis_name="c", subcore_axis_name="s")

  @functools.partial(
      pl.kernel,
      out_type=jax.ShapeDtypeStruct((NC, NROW, d), jnp.float32),
      mesh=mesh,
      scratch_types=[
          pltpu.VMEM((PNCHUNK, PCHUNK), jnp.int32),
          pltpu.VMEM((PNCHUNK, PCHUNK), jnp.int32),
          pltpu.VMEM((PCHUNK, d), jnp.float32),
          pltpu.VMEM((PCHUNK, d), jnp.float32),
          pltpu.VMEM_SHARED((NROW, d), jnp.float32),
          pltpu.SemaphoreType.DMA,
          pltpu.SemaphoreType.DMA,
      ],
      compiler_params=pltpu.CompilerParams(use_tc_tiling_on_sc=False),
  )
  def propagate(g_hbm, src_hbm, dst_hbm, zeros_hbm, out_hbm,
                src_v, dst_v, rows0, rows1, acc, sem0, sem1):
    cid = lax.axis_index("c")
    sid = lax.axis_index("s")
    wid = cid * NS + sid
    pltpu.sync_copy(src_hbm.at[wid], src_v)
    pltpu.sync_copy(dst_hbm.at[wid], dst_v)
    row0 = sid * ROWS_PT
    pltpu.sync_copy(zeros_hbm.at[pl.ds(row0, ROWS_PT)],
                    acc.at[pl.ds(row0, ROWS_PT)])

    @pl.when(sid == 0)
    def _zero_tail():
      pltpu.sync_copy(zeros_hbm.at[pl.ds(NS * ROWS_PT, ROWS_TAIL)],
                      acc.at[pl.ds(NS * ROWS_PT, ROWS_TAIL)])

    plsc.subcore_barrier()

    pltpu.async_copy(g_hbm.at[src_v.at[0]], rows0, sem0)

    @pl.loop(0, PNCHUNK - 1, step=2)
    def _pair(j):
      pltpu.async_copy(g_hbm.at[src_v.at[j + 1]], rows1, sem1)
      pltpu.make_async_copy(g_hbm.at[src_v.at[j]], rows0, sem0).wait()
      pltpu.sync_copy(rows0, acc.at[dst_v.at[j]], add=True)
      pltpu.async_copy(g_hbm.at[src_v.at[j + 2]], rows0, sem0)
      pltpu.make_async_copy(g_hbm.at[src_v.at[j + 1]], rows1, sem1).wait()
      pltpu.sync_copy(rows1, acc.at[dst_v.at[j + 1]], add=True)

    pltpu.make_async_copy(g_hbm.at[src_v.at[PNCHUNK - 1]], rows0, sem0).wait()
    pltpu.sync_copy(rows0, acc.at[dst_v.at[PNCHUNK - 1]], add=True)

    plsc.subcore_barrier()
    pltpu.sync_copy(acc.at[pl.ds(row0, ROWS_PT)],
                    out_hbm.at[cid, pl.ds(row0, ROWS_PT)])

    @pl.when(sid == 0)
    def _copy_tail():
      pltpu.sync_copy(acc.at[pl.ds(NS * ROWS_PT, ROWS_TAIL)],
                      out_hbm.at[cid, pl.ds(NS * ROWS_PT, ROWS_TAIL)])

  return propagate


_PROPAGATE = {d: _make_propagate(d) for d in (128, 64, 48)}


def _make_degree():
  mesh = plsc.VectorSubcoreMesh(core_axis_name="c", subcore_axis_name="s")

  @functools.partial(
      pl.kernel,
      out_type=jax.ShapeDtypeStruct((NC, NROW, DEG_D), jnp.float32),
      mesh=mesh,
      scratch_types=[
          pltpu.VMEM((DEG_NCHUNK, DEG_CHUNK), jnp.int32),
          pltpu.VMEM((DEG_CHUNK, DEG_D), jnp.float32),
          pltpu.VMEM_SHARED((NROW, DEG_D), jnp.float32),
          pltpu.SemaphoreType.DMA,
      ],
      compiler_params=pltpu.CompilerParams(use_tc_tiling_on_sc=False),
  )
  def degree(ones_hbm, dst_hbm, zeros_hbm, out_hbm, dst_v, ones_v, acc, sem):
    cid = lax.axis_index("c")
    sid = lax.axis_index("s")
    wid = cid * NS + sid
    pltpu.sync_copy(dst_hbm.at[wid], dst_v)
    pltpu.sync_copy(ones_hbm, ones_v)
    row0 = sid * ROWS_PT
    pltpu.sync_copy(zeros_hbm.at[pl.ds(row0, ROWS_PT)],
                    acc.at[pl.ds(row0, ROWS_PT)])

    @pl.when(sid == 0)
    def _zero_tail():
      pltpu.sync_copy(zeros_hbm.at[pl.ds(NS * ROWS_PT, ROWS_TAIL)],
                      acc.at[pl.ds(NS * ROWS_PT, ROWS_TAIL)])

    plsc.subcore_barrier()

    @pl.loop(0, DEG_NCHUNK, step=DEG_FIRE)
    def _group(j):
      for k in range(DEG_FIRE):
        pltpu.async_copy(ones_v, acc.at[dst_v.at[j + k]], sem, add=True)
      for k in range(DEG_FIRE):
        pltpu.make_async_copy(ones_v, acc.at[dst_v.at[j + k]], sem).wait()

    plsc.subcore_barrier()
    pltpu.sync_copy(acc.at[pl.ds(row0, ROWS_PT)],
                    out_hbm.at[cid, pl.ds(row0, ROWS_PT)])

    @pl.when(sid == 0)
    def _copy_tail():
      pltpu.sync_copy(acc.at[pl.ds(NS * ROWS_PT, ROWS_TAIL)],
                      out_hbm.at[cid, pl.ds(NS * ROWS_PT, ROWS_TAIL)])

  return degree


_DEGREE = _make_degree()


def _tc_call(body, out_shape, *args):
  return pl.pallas_call(body, out_shape=out_shape)(*args)


def _deg_scale_body(dp_ref, x_ref, dinv_ref, g_ref):
  deg = dp_ref[0, :, 0:1] + dp_ref[1, :, 0:1] + 2.0
  dinv = lax.rsqrt(deg)
  dinv_ref[...] = dinv
  g_ref[0:N, :] = x_ref[...] * dinv[0:N]


def _layer_relu_mm_body(p_ref, g_ref, dinv_ref, w_ref, b_ref, w2_ref, out_ref):
  s = dinv_ref[...] * (p_ref[0] + p_ref[1] + 2.0 * g_ref[...])
  z = jax.nn.relu(
      jnp.dot(s, w_ref[...], preferred_element_type=jnp.float32) + b_ref[...])
  out_ref[...] = dinv_ref[...] * jnp.dot(
      z, w2_ref[...], preferred_element_type=jnp.float32)


def _layer_relu_ew_mm_body(p_ref, g_ref, dinv_ref, b_ref, w2_ref, out_ref):
  z = jax.nn.relu(
      dinv_ref[...] * (p_ref[0] + p_ref[1] + 2.0 * g_ref[...]) + b_ref[...])
  out_ref[...] = dinv_ref[...] * jnp.dot(
      z, w2_ref[...], preferred_element_type=jnp.float32)
  out_ref[N:NROW, :] = jnp.zeros((NROW - N, w2_ref.shape[1]), jnp.float32)


def _layer_lin_mm_body(p_ref, g_ref, dinv_ref, b_ref, w2_ref, out_ref):
  z = dinv_ref[...] * (p_ref[0] + p_ref[1] + 2.0 * g_ref[...]) + b_ref[...]
  out_ref[...] = dinv_ref[...] * jnp.dot(
      z, w2_ref[...], preferred_element_type=jnp.float32)
  out_ref[N:NROW, :] = jnp.zeros((NROW - N, w2_ref.shape[1]), jnp.float32)


def _final_body(p_ref, g_ref, dinv_ref, b_ref, out_ref):
  out_ref[...] = (dinv_ref[...] * (p_ref[0] + p_ref[1] + 2.0 * g_ref[...])
                  + b_ref[...])


def kernel(x, edge_index, W1, b1, W2, b2, W3, b3, W4, b4):
  i32 = jnp.int32
  f32 = jnp.float32
  src_pad = N + jnp.arange(EPAD - E, dtype=i32) % (NROW - N)
  dst_pad = jnp.arange(EPAD - E, dtype=i32)
  src80 = edge_index[0].reshape(NW, 125, 80)
  dst80 = edge_index[1].reshape(NW, 125, 80)
  src128 = jnp.concatenate([edge_index[0], src_pad]).reshape(NW, 79, 128)
  dst128 = jnp.concatenate([edge_index[1], dst_pad]).reshape(NW, 79, 128)
  dst_deg = edge_index[1].reshape(NW, DEG_NCHUNK, DEG_CHUNK)

  zeros_flat = jnp.zeros((NROW * 128,), f32)
  zeros = {d: zeros_flat[:NROW * d].reshape(NROW, d) for d in (128, 64, 48)}
  zeros16 = zeros_flat[:NROW * DEG_D].reshape(NROW, DEG_D)
  ones16 = jnp.ones((DEG_CHUNK, DEG_D), f32)
  W4p = jnp.pad(W4, ((0, 0), (0, 8)))
  b4p = jnp.pad(b4, (0, 8))

  dp = _DEGREE(ones16, dst_deg, zeros16)
  dinv, g0 = _tc_call(
      _deg_scale_body,
      (jax.ShapeDtypeStruct((NROW, 1), f32),
       jax.ShapeDtypeStruct((NROW, 128), f32)),
      dp, x)

  s0 = _PROPAGATE[128](g0, src80, dst80, zeros[128])
  g1 = _tc_call(_layer_relu_mm_body, jax.ShapeDtypeStruct((NROW, 128), f32),
                s0, g0, dinv, W1, b1.reshape(1, -1), W2)

  s1 = _PROPAGATE[128](g1, src80, dst80, zeros[128])
  g2 = _tc_call(_layer_relu_ew_mm_body, jax.ShapeDtypeStruct((NROW, 64), f32),
                s1, g1, dinv, b2.reshape(1, -1), W3)

  s2 = _PROPAGATE[64](g2, src128, dst128, zeros[64])
  g3 = _tc_call(_layer_lin_mm_body, jax.ShapeDtypeStruct((NROW, 48), f32),
                s2, g2, dinv, b3.reshape(1, -1), W4p)

  s3 = _PROPAGATE[48](g3, src128, dst128, zeros[48])
  out = _tc_call(_final_body, jax.ShapeDtypeStruct((NROW, 48), f32),
                 s3, g3, dinv, b4p.reshape(1, -1))
  return out[:N, :40]


# speedup vs baseline: 1.2526x; 1.0011x over previous
import functools

import jax
import jax.numpy as jnp
from jax import lax
from jax.experimental import pallas as pl
from jax.experimental.pallas import tpu as pltpu
from jax.experimental.pallas import tpu_sc as plsc

N = 10000
E = 320000
NC = 2
NS = 16
NW = NC * NS
NROW = N + 128

PCHUNKS = {128: (80, 125, False), 64: (128, 79, False), 48: (128, 79, False)}
EPAD = NW * 79 * 128

DEG_CHUNK = 80
DEG_NCHUNK = 125
DEG_D = 16
DEG_FIRE = 5

ROWS_PT = 624
ROWS_TAIL = NROW - NS * ROWS_PT


def _make_propagate(d):
  PCHUNK, PNCHUNK, tc_tiling = PCHUNKS[d]
  mesh = plsc.VectorSubcoreMesh(core_axis_name="c", subcore_axis_name="s")

  @functools.partial(
      pl.kernel,
      out_type=jax.ShapeDtypeStruct((NC, NROW, d), jnp.float32),
      mesh=mesh,
      scratch_types=[
          pltpu.VMEM((PNCHUNK, PCHUNK), jnp.int32),
          pltpu.VMEM((PNCHUNK, PCHUNK), jnp.int32),
          pltpu.VMEM((PCHUNK, d), jnp.float32),
          pltpu.VMEM((PCHUNK, d), jnp.float32),
          pltpu.VMEM_SHARED((NROW, d), jnp.float32),
          pltpu.SemaphoreType.DMA,
          pltpu.SemaphoreType.DMA,
      ],
      compiler_params=pltpu.CompilerParams(use_tc_tiling_on_sc=tc_tiling),
  )
  def propagate(g_hbm, src_hbm, dst_hbm, zeros_hbm, out_hbm,
                src_v, dst_v, rows0, rows1, acc, sem0, sem1):
    cid = lax.axis_index("c")
    sid = lax.axis_index("s")
    wid = cid * NS + sid
    pltpu.sync_copy(src_hbm.at[wid], src_v)
    pltpu.sync_copy(dst_hbm.at[wid], dst_v)
    row0 = sid * ROWS_PT
    pltpu.sync_copy(zeros_hbm.at[pl.ds(row0, ROWS_PT)],
                    acc.at[pl.ds(row0, ROWS_PT)])

    @pl.when(sid == 0)
    def _zero_tail():
      pltpu.sync_copy(zeros_hbm.at[pl.ds(NS * ROWS_PT, ROWS_TAIL)],
                      acc.at[pl.ds(NS * ROWS_PT, ROWS_TAIL)])

    plsc.subcore_barrier()

    pltpu.async_copy(g_hbm.at[src_v.at[0]], rows0, sem0)

    @pl.loop(0, PNCHUNK - 1, step=2)
    def _pair(j):
      pltpu.async_copy(g_hbm.at[src_v.at[j + 1]], rows1, sem1)
      pltpu.make_async_copy(g_hbm.at[src_v.at[j]], rows0, sem0).wait()
      pltpu.sync_copy(rows0, acc.at[dst_v.at[j]], add=True)
      pltpu.async_copy(g_hbm.at[src_v.at[j + 2]], rows0, sem0)
      pltpu.make_async_copy(g_hbm.at[src_v.at[j + 1]], rows1, sem1).wait()
      pltpu.sync_copy(rows1, acc.at[dst_v.at[j + 1]], add=True)

    pltpu.make_async_copy(g_hbm.at[src_v.at[PNCHUNK - 1]], rows0, sem0).wait()
    pltpu.sync_copy(rows0, acc.at[dst_v.at[PNCHUNK - 1]], add=True)

    plsc.subcore_barrier()
    pltpu.sync_copy(acc.at[pl.ds(row0, ROWS_PT)],
                    out_hbm.at[cid, pl.ds(row0, ROWS_PT)])

    @pl.when(sid == 0)
    def _copy_tail():
      pltpu.sync_copy(acc.at[pl.ds(NS * ROWS_PT, ROWS_TAIL)],
                      out_hbm.at[cid, pl.ds(NS * ROWS_PT, ROWS_TAIL)])

  return propagate


_PROPAGATE = {d: _make_propagate(d) for d in (128, 64, 48)}


def _make_degree():
  mesh = plsc.VectorSubcoreMesh(core_axis_name="c", subcore_axis_name="s")

  @functools.partial(
      pl.kernel,
      out_type=jax.ShapeDtypeStruct((NC, NROW, DEG_D), jnp.float32),
      mesh=mesh,
      scratch_types=[
          pltpu.VMEM((DEG_NCHUNK, DEG_CHUNK), jnp.int32),
          pltpu.VMEM((DEG_CHUNK, DEG_D), jnp.float32),
          pltpu.VMEM_SHARED((NROW, DEG_D), jnp.float32),
          pltpu.SemaphoreType.DMA,
      ],
      compiler_params=pltpu.CompilerParams(use_tc_tiling_on_sc=False),
  )
  def degree(ones_hbm, dst_hbm, zeros_hbm, out_hbm, dst_v, ones_v, acc, sem):
    cid = lax.axis_index("c")
    sid = lax.axis_index("s")
    wid = cid * NS + sid
    pltpu.sync_copy(dst_hbm.at[wid], dst_v)
    pltpu.sync_copy(ones_hbm, ones_v)
    row0 = sid * ROWS_PT
    pltpu.sync_copy(zeros_hbm.at[pl.ds(row0, ROWS_PT)],
                    acc.at[pl.ds(row0, ROWS_PT)])

    @pl.when(sid == 0)
    def _zero_tail():
      pltpu.sync_copy(zeros_hbm.at[pl.ds(NS * ROWS_PT, ROWS_TAIL)],
                      acc.at[pl.ds(NS * ROWS_PT, ROWS_TAIL)])

    plsc.subcore_barrier()

    @pl.loop(0, DEG_NCHUNK, step=DEG_FIRE)
    def _group(j):
      for k in range(DEG_FIRE):
        pltpu.async_copy(ones_v, acc.at[dst_v.at[j + k]], sem, add=True)
      for k in range(DEG_FIRE):
        pltpu.make_async_copy(ones_v, acc.at[dst_v.at[j + k]], sem).wait()

    plsc.subcore_barrier()
    pltpu.sync_copy(acc.at[pl.ds(row0, ROWS_PT)],
                    out_hbm.at[cid, pl.ds(row0, ROWS_PT)])

    @pl.when(sid == 0)
    def _copy_tail():
      pltpu.sync_copy(acc.at[pl.ds(NS * ROWS_PT, ROWS_TAIL)],
                      out_hbm.at[cid, pl.ds(NS * ROWS_PT, ROWS_TAIL)])

  return degree


_DEGREE = _make_degree()


def _tc_call(body, out_shape, *args):
  return pl.pallas_call(body, out_shape=out_shape)(*args)


def _deg_scale_body(dp_ref, x_ref, dinv_ref, g_ref):
  deg = dp_ref[0, :, 0:1] + dp_ref[1, :, 0:1] + 2.0
  dinv = lax.rsqrt(deg)
  dinv_ref[...] = dinv
  g_ref[0:N, :] = x_ref[...] * dinv[0:N]


def _layer_relu_mm_body(p_ref, g_ref, dinv_ref, w_ref, b_ref, w2_ref, out_ref):
  s = dinv_ref[...] * (p_ref[0] + p_ref[1] + 2.0 * g_ref[...])
  z = jax.nn.relu(
      jnp.dot(s, w_ref[...], preferred_element_type=jnp.float32) + b_ref[...])
  out_ref[...] = dinv_ref[...] * jnp.dot(
      z, w2_ref[...], preferred_element_type=jnp.float32)


def _layer_relu_ew_mm_body(p_ref, g_ref, dinv_ref, b_ref, w2_ref, out_ref):
  z = jax.nn.relu(
      dinv_ref[...] * (p_ref[0] + p_ref[1] + 2.0 * g_ref[...]) + b_ref[...])
  out_ref[...] = dinv_ref[...] * jnp.dot(
      z, w2_ref[...], preferred_element_type=jnp.float32)
  out_ref[N:NROW, :] = jnp.zeros((NROW - N, w2_ref.shape[1]), jnp.float32)


def _layer_lin_mm_body(p_ref, g_ref, dinv_ref, b_ref, w2_ref, out_ref):
  z = dinv_ref[...] * (p_ref[0] + p_ref[1] + 2.0 * g_ref[...]) + b_ref[...]
  out_ref[...] = dinv_ref[...] * jnp.dot(
      z, w2_ref[...], preferred_element_type=jnp.float32)
  out_ref[N:NROW, :] = jnp.zeros((NROW - N, w2_ref.shape[1]), jnp.float32)


def _final_body(p_ref, g_ref, dinv_ref, b_ref, out_ref):
  v = (dinv_ref[...] * (p_ref[0] + p_ref[1] + 2.0 * g_ref[...]) + b_ref[...])
  out_ref[...] = v[0:N, 0:40]


def kernel(x, edge_index, W1, b1, W2, b2, W3, b3, W4, b4):
  i32 = jnp.int32
  f32 = jnp.float32
  src_pad = N + jnp.arange(EPAD - E, dtype=i32) % (NROW - N)
  dst_pad = jnp.arange(EPAD - E, dtype=i32)
  src80 = edge_index[0].reshape(NW, 125, 80)
  dst80 = edge_index[1].reshape(NW, 125, 80)
  src128 = jnp.concatenate([edge_index[0], src_pad]).reshape(NW, 79, 128)
  dst128 = jnp.concatenate([edge_index[1], dst_pad]).reshape(NW, 79, 128)
  dst_deg = edge_index[1].reshape(NW, DEG_NCHUNK, DEG_CHUNK)

  zeros = {d: jnp.zeros((NROW, d), f32) for d in (128, 64, 48)}
  zeros16 = jnp.zeros((NROW, DEG_D), f32)
  ones16 = jnp.ones((DEG_CHUNK, DEG_D), f32)
  W4p = jnp.pad(W4, ((0, 0), (0, 8)))
  b4p = jnp.pad(b4, (0, 8))

  dp = _DEGREE(ones16, dst_deg, zeros16)
  dinv, g0 = _tc_call(
      _deg_scale_body,
      (jax.ShapeDtypeStruct((NROW, 1), f32),
       jax.ShapeDtypeStruct((NROW, 128), f32)),
      dp, x)

  s0 = _PROPAGATE[128](g0, src80, dst80, zeros[128])
  g1 = _tc_call(_layer_relu_mm_body, jax.ShapeDtypeStruct((NROW, 128), f32),
                s0, g0, dinv, W1, b1.reshape(1, -1), W2)

  s1 = _PROPAGATE[128](g1, src80, dst80, zeros[128])
  g2 = _tc_call(_layer_relu_ew_mm_body, jax.ShapeDtypeStruct((NROW, 64), f32),
                s1, g1, dinv, b2.reshape(1, -1), W3)

  s2 = _PROPAGATE[64](g2, src128, dst128, zeros[64])
  g3 = _tc_call(_layer_lin_mm_body, jax.ShapeDtypeStruct((NROW, 48), f32),
                s2, g2, dinv, b3.reshape(1, -1), W4p)

  s3 = _PROPAGATE[48](g3, src128, dst128, zeros[48])
  return _tc_call(_final_body, jax.ShapeDtypeStruct((N, 40), f32),
                  s3, g3, dinv, b4p.reshape(1, -1))


# speedup vs baseline: 1.4621x; 1.1672x over previous
import functools

import jax
import jax.numpy as jnp
from jax import lax
from jax.experimental import pallas as pl
from jax.experimental.pallas import tpu as pltpu
from jax.experimental.pallas import tpu_sc as plsc

N = 10000
E = 320000
NC = 2
NS = 16
NW = NC * NS
NRJ = N + 128
NRA = N + 8

PCHUNKS = {128: (80, 125, False), 64: (128, 79, True), 48: (128, 79, True)}
EPAD = NW * 79 * 128

DEG_CHUNK = 80
DEG_NCHUNK = 125
DEG_D = 16
DEG_FIRE = 5

ROWS_PT = 624
ROWS_TAIL = NRA - NS * ROWS_PT

NBUF = 3


def _make_propagate(d):
  PCHUNK, PNCHUNK, _ = PCHUNKS[d]
  mesh = plsc.VectorSubcoreMesh(core_axis_name="c", subcore_axis_name="s")

  rows_bufs = [pltpu.VMEM((PCHUNK, d), jnp.float32) for _ in range(NBUF)]
  sems = [pltpu.SemaphoreType.DMA for _ in range(NBUF)]

  @functools.partial(
      pl.kernel,
      out_type=jax.ShapeDtypeStruct((NC, NRA, d), jnp.float32),
      mesh=mesh,
      scratch_types=[
          pltpu.VMEM((PNCHUNK, PCHUNK), jnp.int32),
          pltpu.VMEM((PNCHUNK, PCHUNK), jnp.int32),
          pltpu.VMEM_SHARED((NRA, d), jnp.float32),
      ] + rows_bufs + sems,
      compiler_params=pltpu.CompilerParams(use_tc_tiling_on_sc=False),
  )
  def propagate(g_hbm, src_hbm, dst_hbm, zeros_hbm, out_hbm,
                src_v, dst_v, acc, *bufs_sems):
    rows = bufs_sems[:NBUF]
    sem = bufs_sems[NBUF:]
    cid = lax.axis_index("c")
    sid = lax.axis_index("s")
    wid = cid * NS + sid
    pltpu.sync_copy(src_hbm.at[wid], src_v)
    pltpu.sync_copy(dst_hbm.at[wid], dst_v)
    row0 = sid * ROWS_PT
    pltpu.sync_copy(zeros_hbm.at[pl.ds(row0, ROWS_PT)],
                    acc.at[pl.ds(row0, ROWS_PT)])

    @pl.when(sid == 0)
    def _zero_tail():
      pltpu.sync_copy(zeros_hbm.at[pl.ds(NS * ROWS_PT, ROWS_TAIL)],
                      acc.at[pl.ds(NS * ROWS_PT, ROWS_TAIL)])

    plsc.subcore_barrier()

    for c in range(NBUF):
      pltpu.async_copy(g_hbm.at[src_v.at[c]], rows[c], sem[c])

    loop_end = (PNCHUNK // NBUF) * NBUF

    @pl.loop(0, loop_end, step=NBUF)
    def _group(j):
      for b in range(NBUF):
        pltpu.make_async_copy(g_hbm.at[src_v.at[j + b]], rows[b],
                              sem[b]).wait()
        pltpu.sync_copy(rows[b], acc.at[dst_v.at[j + b]], add=True)

        @pl.when(j + b + NBUF < PNCHUNK)
        def _next():
          pltpu.async_copy(g_hbm.at[src_v.at[j + b + NBUF]], rows[b], sem[b])

    for c in range(loop_end, PNCHUNK):
      b = c % NBUF
      pltpu.make_async_copy(g_hbm.at[src_v.at[c]], rows[b], sem[b]).wait()
      pltpu.sync_copy(rows[b], acc.at[dst_v.at[c]], add=True)

    plsc.subcore_barrier()
    pltpu.sync_copy(acc.at[pl.ds(row0, ROWS_PT)],
                    out_hbm.at[cid, pl.ds(row0, ROWS_PT)])

    @pl.when(sid == 0)
    def _copy_tail():
      pltpu.sync_copy(acc.at[pl.ds(NS * ROWS_PT, ROWS_TAIL)],
                      out_hbm.at[cid, pl.ds(NS * ROWS_PT, ROWS_TAIL)])

  return propagate


_PROPAGATE = {d: _make_propagate(d) for d in (128, 64, 48)}


def _make_degree():
  mesh = plsc.VectorSubcoreMesh(core_axis_name="c", subcore_axis_name="s")

  @functools.partial(
      pl.kernel,
      out_type=jax.ShapeDtypeStruct((NC, NRA, DEG_D), jnp.float32),
      mesh=mesh,
      scratch_types=[
          pltpu.VMEM((DEG_NCHUNK, DEG_CHUNK), jnp.int32),
          pltpu.VMEM((DEG_CHUNK, DEG_D), jnp.float32),
          pltpu.VMEM_SHARED((NRA, DEG_D), jnp.float32),
          pltpu.SemaphoreType.DMA,
      ],
      compiler_params=pltpu.CompilerParams(use_tc_tiling_on_sc=False),
  )
  def degree(ones_hbm, dst_hbm, zeros_hbm, out_hbm, dst_v, ones_v, acc, sem):
    cid = lax.axis_index("c")
    sid = lax.axis_index("s")
    wid = cid * NS + sid
    pltpu.sync_copy(dst_hbm.at[wid], dst_v)
    pltpu.sync_copy(ones_hbm, ones_v)
    row0 = sid * ROWS_PT
    pltpu.sync_copy(zeros_hbm.at[pl.ds(row0, ROWS_PT)],
                    acc.at[pl.ds(row0, ROWS_PT)])

    @pl.when(sid == 0)
    def _zero_tail():
      pltpu.sync_copy(zeros_hbm.at[pl.ds(NS * ROWS_PT, ROWS_TAIL)],
                      acc.at[pl.ds(NS * ROWS_PT, ROWS_TAIL)])

    plsc.subcore_barrier()

    @pl.loop(0, DEG_NCHUNK, step=DEG_FIRE)
    def _group(j):
      for k in range(DEG_FIRE):
        pltpu.async_copy(ones_v, acc.at[dst_v.at[j + k]], sem, add=True)
      for k in range(DEG_FIRE):
        pltpu.make_async_copy(ones_v, acc.at[dst_v.at[j + k]], sem).wait()

    plsc.subcore_barrier()
    pltpu.sync_copy(acc.at[pl.ds(row0, ROWS_PT)],
                    out_hbm.at[cid, pl.ds(row0, ROWS_PT)])

    @pl.when(sid == 0)
    def _copy_tail():
      pltpu.sync_copy(acc.at[pl.ds(NS * ROWS_PT, ROWS_TAIL)],
                      out_hbm.at[cid, pl.ds(NS * ROWS_PT, ROWS_TAIL)])

  return degree


_DEGREE = _make_degree()


def _tc_call(body, out_shape, *args):
  return pl.pallas_call(body, out_shape=out_shape)(*args)


def _deg_scale_body(dp_ref, x_ref, dinv_ref, g_ref):
  deg = dp_ref[0, 0:N, 0:1] + dp_ref[1, 0:N, 0:1] + 2.0
  dinv = lax.rsqrt(deg)
  dinv_ref[...] = dinv
  g_ref[...] = x_ref[...] * dinv


def _layer_relu_mm_body(p_ref, g_ref, dinv_ref, w_ref, b_ref, w2_ref, out_ref):
  s = dinv_ref[...] * (p_ref[0, 0:N] + p_ref[1, 0:N] + 2.0 * g_ref[...])
  z = jax.nn.relu(
      jnp.dot(s, w_ref[...], preferred_element_type=jnp.float32) + b_ref[...])
  out_ref[...] = dinv_ref[...] * jnp.dot(
      z, w2_ref[...], preferred_element_type=jnp.float32)


def _layer_relu_ew_mm_body(p_ref, g_ref, dinv_ref, b_ref, w2_ref, out_ref):
  z = jax.nn.relu(
      dinv_ref[...] * (p_ref[0, 0:N] + p_ref[1, 0:N] + 2.0 * g_ref[...])
      + b_ref[...])
  out_ref[0:N, :] = dinv_ref[...] * jnp.dot(
      z, w2_ref[...], preferred_element_type=jnp.float32)
  out_ref[N:NRJ, :] = jnp.zeros((NRJ - N, w2_ref.shape[1]), jnp.float32)


def _layer_lin_mm_body(p_ref, g_ref, dinv_ref, b_ref, w2_ref, out_ref):
  z = (dinv_ref[...] * (p_ref[0, 0:N] + p_ref[1, 0:N] + 2.0 * g_ref[0:N])
       + b_ref[...])
  out_ref[0:N, :] = dinv_ref[...] * jnp.dot(
      z, w2_ref[...], preferred_element_type=jnp.float32)
  out_ref[N:NRJ, :] = jnp.zeros((NRJ - N, w2_ref.shape[1]), jnp.float32)


def _final_body(p_ref, g_ref, dinv_ref, b_ref, out_ref):
  v = (dinv_ref[...] * (p_ref[0, 0:N] + p_ref[1, 0:N] + 2.0 * g_ref[0:N])
       + b_ref[...])
  out_ref[...] = v[:, 0:40]


def kernel(x, edge_index, W1, b1, W2, b2, W3, b3, W4, b4):
  i32 = jnp.int32
  f32 = jnp.float32
  src_pad = N + jnp.arange(EPAD - E, dtype=i32) % (NRJ - N)
  dst_pad = jnp.arange(EPAD - E, dtype=i32)
  src80 = edge_index[0].reshape(NW, 125, 80)
  dst80 = edge_index[1].reshape(NW, 125, 80)
  src128 = jnp.concatenate([edge_index[0], src_pad]).reshape(NW, 79, 128)
  dst128 = jnp.concatenate([edge_index[1], dst_pad]).reshape(NW, 79, 128)
  dst_deg = edge_index[1].reshape(NW, DEG_NCHUNK, DEG_CHUNK)

  zeros = {d: jnp.zeros((NRA, d), f32) for d in (128, 64, 48)}
  zeros16 = jnp.zeros((NRA, DEG_D), f32)
  ones16 = jnp.ones((DEG_CHUNK, DEG_D), f32)
  W4p = jnp.pad(W4, ((0, 0), (0, 8)))
  b4p = jnp.pad(b4, (0, 8))

  dp = _DEGREE(ones16, dst_deg, zeros16)
  dinv, g0 = _tc_call(
      _deg_scale_body,
      (jax.ShapeDtypeStruct((N, 1), f32),
       jax.ShapeDtypeStruct((N, 128), f32)),
      dp, x)

  s0 = _PROPAGATE[128](g0, src80, dst80, zeros[128])
  g1 = _tc_call(_layer_relu_mm_body, jax.ShapeDtypeStruct((N, 128), f32),
                s0, g0, dinv, W1, b1.reshape(1, -1), W2)

  s1 = _PROPAGATE[128](g1, src80, dst80, zeros[128])
  g2 = _tc_call(_layer_relu_ew_mm_body, jax.ShapeDtypeStruct((NRJ, 64), f32),
                s1, g1, dinv, b2.reshape(1, -1), W3)

  s2 = _PROPAGATE[64](g2, src128, dst128, zeros[64])
  g3 = _tc_call(_layer_lin_mm_body, jax.ShapeDtypeStruct((NRJ, 48), f32),
                s2, g2, dinv, b3.reshape(1, -1), W4p)

  s3 = _PROPAGATE[48](g3, src128, dst128, zeros[48])
  return _tc_call(_final_body, jax.ShapeDtypeStruct((N, 40), f32),
                  s3, g3, dinv, b4p.reshape(1, -1))


# speedup vs baseline: 1.4846x; 1.0154x over previous
import functools

import jax
import jax.numpy as jnp
from jax import lax
from jax.experimental import pallas as pl
from jax.experimental.pallas import tpu as pltpu
from jax.experimental.pallas import tpu_sc as plsc

N = 10000
E = 320000
NC = 2
NS = 16
NW = NC * NS
NRJ = N + 128
NRA = N + 8

PCHUNKS = {128: (80, 125, 3), 64: (128, 79, 4), 48: (128, 79, 4)}
EPAD = NW * 79 * 128

DEG_CHUNK = 80
DEG_NCHUNK = 125
DEG_D = 16
DEG_FIRE = 5

ROWS_PT = 624
ROWS_TAIL = NRA - NS * ROWS_PT


def _make_propagate(d):
  PCHUNK, PNCHUNK, NBUF = PCHUNKS[d]
  mesh = plsc.VectorSubcoreMesh(core_axis_name="c", subcore_axis_name="s")

  rows_bufs = [pltpu.VMEM((PCHUNK, d), jnp.float32) for _ in range(NBUF)]
  sems = [pltpu.SemaphoreType.DMA for _ in range(NBUF)]

  @functools.partial(
      pl.kernel,
      out_type=jax.ShapeDtypeStruct((NC, NRA, d), jnp.float32),
      mesh=mesh,
      scratch_types=[
          pltpu.VMEM((PNCHUNK, PCHUNK), jnp.int32),
          pltpu.VMEM((PNCHUNK, PCHUNK), jnp.int32),
          pltpu.VMEM_SHARED((NRA, d), jnp.float32),
      ] + rows_bufs + sems,
      compiler_params=pltpu.CompilerParams(use_tc_tiling_on_sc=False),
  )
  def propagate(g_hbm, src_hbm, dst_hbm, zeros_hbm, out_hbm,
                src_v, dst_v, acc, *bufs_sems):
    rows = bufs_sems[:NBUF]
    sem = bufs_sems[NBUF:]
    cid = lax.axis_index("c")
    sid = lax.axis_index("s")
    wid = cid * NS + sid
    pltpu.sync_copy(src_hbm.at[wid], src_v)
    pltpu.sync_copy(dst_hbm.at[wid], dst_v)
    row0 = sid * ROWS_PT
    pltpu.sync_copy(zeros_hbm.at[pl.ds(row0, ROWS_PT)],
                    acc.at[pl.ds(row0, ROWS_PT)])

    @pl.when(sid == 0)
    def _zero_tail():
      pltpu.sync_copy(zeros_hbm.at[pl.ds(NS * ROWS_PT, ROWS_TAIL)],
                      acc.at[pl.ds(NS * ROWS_PT, ROWS_TAIL)])

    plsc.subcore_barrier()

    for c in range(NBUF):
      pltpu.async_copy(g_hbm.at[src_v.at[c]], rows[c], sem[c])

    loop_end = (PNCHUNK // NBUF) * NBUF

    @pl.loop(0, loop_end, step=NBUF)
    def _group(j):
      for b in range(NBUF):
        pltpu.make_async_copy(g_hbm.at[src_v.at[j + b]], rows[b],
                              sem[b]).wait()
        pltpu.sync_copy(rows[b], acc.at[dst_v.at[j + b]], add=True)

        @pl.when(j + b + NBUF < PNCHUNK)
        def _next():
          pltpu.async_copy(g_hbm.at[src_v.at[j + b + NBUF]], rows[b], sem[b])

    for c in range(loop_end, PNCHUNK):
      b = c % NBUF
      pltpu.make_async_copy(g_hbm.at[src_v.at[c]], rows[b], sem[b]).wait()
      pltpu.sync_copy(rows[b], acc.at[dst_v.at[c]], add=True)

    plsc.subcore_barrier()
    pltpu.sync_copy(acc.at[pl.ds(row0, ROWS_PT)],
                    out_hbm.at[cid, pl.ds(row0, ROWS_PT)])

    @pl.when(sid == 0)
    def _copy_tail():
      pltpu.sync_copy(acc.at[pl.ds(NS * ROWS_PT, ROWS_TAIL)],
                      out_hbm.at[cid, pl.ds(NS * ROWS_PT, ROWS_TAIL)])

  return propagate


_PROPAGATE = {d: _make_propagate(d) for d in (128, 64, 48)}


def _make_degree():
  mesh = plsc.VectorSubcoreMesh(core_axis_name="c", subcore_axis_name="s")

  @functools.partial(
      pl.kernel,
      out_type=jax.ShapeDtypeStruct((NC, NRA, DEG_D), jnp.float32),
      mesh=mesh,
      scratch_types=[
          pltpu.VMEM((DEG_NCHUNK, DEG_CHUNK), jnp.int32),
          pltpu.VMEM((DEG_CHUNK, DEG_D), jnp.float32),
          pltpu.VMEM_SHARED((NRA, DEG_D), jnp.float32),
          pltpu.SemaphoreType.DMA,
      ],
      compiler_params=pltpu.CompilerParams(use_tc_tiling_on_sc=False),
  )
  def degree(ones_hbm, dst_hbm, zeros_hbm, out_hbm, dst_v, ones_v, acc, sem):
    cid = lax.axis_index("c")
    sid = lax.axis_index("s")
    wid = cid * NS + sid
    pltpu.sync_copy(dst_hbm.at[wid], dst_v)
    pltpu.sync_copy(ones_hbm, ones_v)
    row0 = sid * ROWS_PT
    pltpu.sync_copy(zeros_hbm.at[pl.ds(row0, ROWS_PT)],
                    acc.at[pl.ds(row0, ROWS_PT)])

    @pl.when(sid == 0)
    def _zero_tail():
      pltpu.sync_copy(zeros_hbm.at[pl.ds(NS * ROWS_PT, ROWS_TAIL)],
                      acc.at[pl.ds(NS * ROWS_PT, ROWS_TAIL)])

    plsc.subcore_barrier()

    @pl.loop(0, DEG_NCHUNK, step=DEG_FIRE)
    def _group(j):
      for k in range(DEG_FIRE):
        pltpu.async_copy(ones_v, acc.at[dst_v.at[j + k]], sem, add=True)
      for k in range(DEG_FIRE):
        pltpu.make_async_copy(ones_v, acc.at[dst_v.at[j + k]], sem).wait()

    plsc.subcore_barrier()
    pltpu.sync_copy(acc.at[pl.ds(row0, ROWS_PT)],
                    out_hbm.at[cid, pl.ds(row0, ROWS_PT)])

    @pl.when(sid == 0)
    def _copy_tail():
      pltpu.sync_copy(acc.at[pl.ds(NS * ROWS_PT, ROWS_TAIL)],
                      out_hbm.at[cid, pl.ds(NS * ROWS_PT, ROWS_TAIL)])

  return degree


_DEGREE = _make_degree()


def _tc_call(body, out_shape, *args):
  return pl.pallas_call(body, out_shape=out_shape)(*args)


def _deg_scale_body(dp_ref, x_ref, dinv_ref, g_ref):
  deg = dp_ref[0, 0:N, 0:1] + dp_ref[1, 0:N, 0:1] + 2.0
  dinv = lax.rsqrt(deg)
  dinv_ref[...] = dinv
  g_ref[...] = x_ref[...] * dinv


def _layer_relu_mm_body(p_ref, g_ref, dinv_ref, w_ref, b_ref, w2_ref, out_ref):
  s = dinv_ref[...] * (p_ref[0, 0:N] + p_ref[1, 0:N] + 2.0 * g_ref[...])
  z = jax.nn.relu(
      jnp.dot(s, w_ref[...], preferred_element_type=jnp.float32) + b_ref[...])
  out_ref[...] = dinv_ref[...] * jnp.dot(
      z, w2_ref[...], preferred_element_type=jnp.float32)


def _layer_relu_ew_mm_body(p_ref, g_ref, dinv_ref, b_ref, w2_ref, out_ref):
  z = jax.nn.relu(
      dinv_ref[...] * (p_ref[0, 0:N] + p_ref[1, 0:N] + 2.0 * g_ref[...])
      + b_ref[...])
  out_ref[0:N, :] = dinv_ref[...] * jnp.dot(
      z, w2_ref[...], preferred_element_type=jnp.float32)
  out_ref[N:NRJ, :] = jnp.zeros((NRJ - N, w2_ref.shape[1]), jnp.float32)


def _layer_lin_mm_body(p_ref, g_ref, dinv_ref, b_ref, w2_ref, out_ref):
  z = (dinv_ref[...] * (p_ref[0, 0:N] + p_ref[1, 0:N] + 2.0 * g_ref[0:N])
       + b_ref[...])
  out_ref[0:N, :] = dinv_ref[...] * jnp.dot(
      z, w2_ref[...], preferred_element_type=jnp.float32)
  out_ref[N:NRJ, :] = jnp.zeros((NRJ - N, w2_ref.shape[1]), jnp.float32)


def _final_body(p_ref, g_ref, dinv_ref, b_ref, out_ref):
  v = (dinv_ref[...] * (p_ref[0, 0:N] + p_ref[1, 0:N] + 2.0 * g_ref[0:N])
       + b_ref[...])
  out_ref[...] = v[:, 0:40]


def kernel(x, edge_index, W1, b1, W2, b2, W3, b3, W4, b4):
  i32 = jnp.int32
  f32 = jnp.float32
  src_pad = N + jnp.arange(EPAD - E, dtype=i32) % (NRJ - N)
  dst_pad = jnp.arange(EPAD - E, dtype=i32)
  src80 = edge_index[0].reshape(NW, 125, 80)
  dst80 = edge_index[1].reshape(NW, 125, 80)
  src128 = jnp.concatenate([edge_index[0], src_pad]).reshape(NW, 79, 128)
  dst128 = jnp.concatenate([edge_index[1], dst_pad]).reshape(NW, 79, 128)
  dst_deg = edge_index[1].reshape(NW, DEG_NCHUNK, DEG_CHUNK)

  zeros = {d: jnp.zeros((NRA, d), f32) for d in (128, 64, 48)}
  zeros16 = jnp.zeros((NRA, DEG_D), f32)
  ones16 = jnp.ones((DEG_CHUNK, DEG_D), f32)
  W4p = jnp.pad(W4, ((0, 0), (0, 8)))
  b4p = jnp.pad(b4, (0, 8))

  dp = _DEGREE(ones16, dst_deg, zeros16)
  dinv, g0 = _tc_call(
      _deg_scale_body,
      (jax.ShapeDtypeStruct((N, 1), f32),
       jax.ShapeDtypeStruct((N, 128), f32)),
      dp, x)

  s0 = _PROPAGATE[128](g0, src80, dst80, zeros[128])
  g1 = _tc_call(_layer_relu_mm_body, jax.ShapeDtypeStruct((N, 128), f32),
                s0, g0, dinv, W1, b1.reshape(1, -1), W2)

  s1 = _PROPAGATE[128](g1, src80, dst80, zeros[128])
  g2 = _tc_call(_layer_relu_ew_mm_body, jax.ShapeDtypeStruct((NRJ, 64), f32),
                s1, g1, dinv, b2.reshape(1, -1), W3)

  s2 = _PROPAGATE[64](g2, src128, dst128, zeros[64])
  g3 = _tc_call(_layer_lin_mm_body, jax.ShapeDtypeStruct((NRJ, 48), f32),
                s2, g2, dinv, b3.reshape(1, -1), W4p)

  s3 = _PROPAGATE[48](g3, src128, dst128, zeros[48])
  return _tc_call(_final_body, jax.ShapeDtypeStruct((N, 40), f32),
                  s3, g3, dinv, b4p.reshape(1, -1))


# speedup vs baseline: 1.5433x; 1.0396x over previous
import functools

import jax
import jax.numpy as jnp
from jax import lax
from jax.experimental import pallas as pl
from jax.experimental.pallas import tpu as pltpu
from jax.experimental.pallas import tpu_sc as plsc

N = 10000
E = 320000
NC = 2
NS = 16
NW = NC * NS
NRJ = N + 128
NRA = N + 8

PCHUNKS = {128: (80, 125, 3), 64: (128, 79, 4), 48: (128, 79, 4)}
EPAD = NW * 79 * 128

DEG_CHUNK = 80
DEG_NCHUNK = 125
DEG_D = 16
DEG_FIRE = 5

ROWS_PT = 624
ROWS_TAIL = NRA - NS * ROWS_PT


def _make_propagate(d):
  PCHUNK, PNCHUNK, NBUF = PCHUNKS[d]
  mesh = plsc.VectorSubcoreMesh(core_axis_name="c", subcore_axis_name="s")

  rows_bufs = [pltpu.VMEM((PCHUNK, d), jnp.float32) for _ in range(NBUF)]
  sems = [pltpu.SemaphoreType.DMA for _ in range(NBUF)]

  paired = d < 128
  out_shape = (NRA, 128) if paired else (NC, NRA, d)

  @functools.partial(
      pl.kernel,
      out_type=jax.ShapeDtypeStruct(out_shape, jnp.float32),
      mesh=mesh,
      scratch_types=[
          pltpu.VMEM((PNCHUNK, PCHUNK), jnp.int32),
          pltpu.VMEM((PNCHUNK, PCHUNK), jnp.int32),
          pltpu.VMEM_SHARED((NRA, d), jnp.float32),
      ] + rows_bufs + sems,
      compiler_params=pltpu.CompilerParams(use_tc_tiling_on_sc=False),
  )
  def propagate(g_hbm, src_hbm, dst_hbm, zeros_hbm, out_hbm,
                src_v, dst_v, acc, *bufs_sems):
    rows = bufs_sems[:NBUF]
    sem = bufs_sems[NBUF:]
    cid = lax.axis_index("c")
    sid = lax.axis_index("s")
    wid = cid * NS + sid
    pltpu.sync_copy(src_hbm.at[wid], src_v)
    pltpu.sync_copy(dst_hbm.at[wid], dst_v)
    row0 = sid * ROWS_PT
    pltpu.sync_copy(zeros_hbm.at[pl.ds(row0, ROWS_PT)],
                    acc.at[pl.ds(row0, ROWS_PT)])

    @pl.when(sid == 0)
    def _zero_tail():
      pltpu.sync_copy(zeros_hbm.at[pl.ds(NS * ROWS_PT, ROWS_TAIL)],
                      acc.at[pl.ds(NS * ROWS_PT, ROWS_TAIL)])

    plsc.subcore_barrier()

    for c in range(NBUF):
      pltpu.async_copy(g_hbm.at[src_v.at[c]], rows[c], sem[c])

    loop_end = (PNCHUNK // NBUF) * NBUF

    @pl.loop(0, loop_end, step=NBUF)
    def _group(j):
      for b in range(NBUF):
        pltpu.make_async_copy(g_hbm.at[src_v.at[j + b]], rows[b],
                              sem[b]).wait()
        pltpu.sync_copy(rows[b], acc.at[dst_v.at[j + b]], add=True)

        @pl.when(j + b + NBUF < PNCHUNK)
        def _next():
          pltpu.async_copy(g_hbm.at[src_v.at[j + b + NBUF]], rows[b], sem[b])

    for c in range(loop_end, PNCHUNK):
      b = c % NBUF
      pltpu.make_async_copy(g_hbm.at[src_v.at[c]], rows[b], sem[b]).wait()
      pltpu.sync_copy(rows[b], acc.at[dst_v.at[c]], add=True)

    plsc.subcore_barrier()
    if paired:
      pltpu.sync_copy(acc.at[pl.ds(row0, ROWS_PT)],
                      out_hbm.at[pl.ds(row0, ROWS_PT), pl.ds(cid * d, d)])

      @pl.when(sid == 0)
      def _copy_tail():
        pltpu.sync_copy(acc.at[pl.ds(NS * ROWS_PT, ROWS_TAIL)],
                        out_hbm.at[pl.ds(NS * ROWS_PT, ROWS_TAIL),
                                   pl.ds(cid * d, d)])
    else:
      pltpu.sync_copy(acc.at[pl.ds(row0, ROWS_PT)],
                      out_hbm.at[cid, pl.ds(row0, ROWS_PT)])

      @pl.when(sid == 0)
      def _copy_tail():
        pltpu.sync_copy(acc.at[pl.ds(NS * ROWS_PT, ROWS_TAIL)],
                        out_hbm.at[cid, pl.ds(NS * ROWS_PT, ROWS_TAIL)])

  return propagate


_PROPAGATE = {d: _make_propagate(d) for d in (128, 64, 48)}


def _make_degree():
  mesh = plsc.VectorSubcoreMesh(core_axis_name="c", subcore_axis_name="s")

  @functools.partial(
      pl.kernel,
      out_type=jax.ShapeDtypeStruct((NC, NRA, DEG_D), jnp.float32),
      mesh=mesh,
      scratch_types=[
          pltpu.VMEM((DEG_NCHUNK, DEG_CHUNK), jnp.int32),
          pltpu.VMEM((DEG_CHUNK, DEG_D), jnp.float32),
          pltpu.VMEM_SHARED((NRA, DEG_D), jnp.float32),
          pltpu.SemaphoreType.DMA,
      ],
      compiler_params=pltpu.CompilerParams(use_tc_tiling_on_sc=False),
  )
  def degree(ones_hbm, dst_hbm, zeros_hbm, out_hbm, dst_v, ones_v, acc, sem):
    cid = lax.axis_index("c")
    sid = lax.axis_index("s")
    wid = cid * NS + sid
    pltpu.sync_copy(dst_hbm.at[wid], dst_v)
    pltpu.sync_copy(ones_hbm, ones_v)
    row0 = sid * ROWS_PT
    pltpu.sync_copy(zeros_hbm.at[pl.ds(row0, ROWS_PT)],
                    acc.at[pl.ds(row0, ROWS_PT)])

    @pl.when(sid == 0)
    def _zero_tail():
      pltpu.sync_copy(zeros_hbm.at[pl.ds(NS * ROWS_PT, ROWS_TAIL)],
                      acc.at[pl.ds(NS * ROWS_PT, ROWS_TAIL)])

    plsc.subcore_barrier()

    @pl.loop(0, DEG_NCHUNK, step=DEG_FIRE)
    def _group(j):
      for k in range(DEG_FIRE):
        pltpu.async_copy(ones_v, acc.at[dst_v.at[j + k]], sem, add=True)
      for k in range(DEG_FIRE):
        pltpu.make_async_copy(ones_v, acc.at[dst_v.at[j + k]], sem).wait()

    plsc.subcore_barrier()
    pltpu.sync_copy(acc.at[pl.ds(row0, ROWS_PT)],
                    out_hbm.at[cid, pl.ds(row0, ROWS_PT)])

    @pl.when(sid == 0)
    def _copy_tail():
      pltpu.sync_copy(acc.at[pl.ds(NS * ROWS_PT, ROWS_TAIL)],
                      out_hbm.at[cid, pl.ds(NS * ROWS_PT, ROWS_TAIL)])

  return degree


_DEGREE = _make_degree()


def _tc_call(body, out_shape, *args):
  return pl.pallas_call(body, out_shape=out_shape)(*args)


def _deg_scale_body(dp_ref, x_ref, dinv_ref, g_ref):
  deg = dp_ref[0, 0:N, 0:1] + dp_ref[1, 0:N, 0:1] + 2.0
  dinv = lax.rsqrt(deg)
  dinv_ref[...] = dinv
  g_ref[...] = x_ref[...] * dinv


def _layer_relu_mm_body(p_ref, g_ref, dinv_ref, w_ref, b_ref, w2_ref, out_ref):
  s = dinv_ref[...] * (p_ref[0, 0:N] + p_ref[1, 0:N] + 2.0 * g_ref[...])
  z = jax.nn.relu(
      jnp.dot(s, w_ref[...], preferred_element_type=jnp.float32) + b_ref[...])
  out_ref[...] = dinv_ref[...] * jnp.dot(
      z, w2_ref[...], preferred_element_type=jnp.float32)


def _layer_relu_ew_mm_body(p_ref, g_ref, dinv_ref, b_ref, w2_ref, out_ref):
  z = jax.nn.relu(
      dinv_ref[...] * (p_ref[0, 0:N] + p_ref[1, 0:N] + 2.0 * g_ref[...])
      + b_ref[...])
  out_ref[0:N, :] = dinv_ref[...] * jnp.dot(
      z, w2_ref[...], preferred_element_type=jnp.float32)
  out_ref[N:NRJ, :] = jnp.zeros((NRJ - N, w2_ref.shape[1]), jnp.float32)


def _layer_lin_mm_body(p_ref, g_ref, dinv_ref, b_ref, w2_ref, out_ref):
  z = (dinv_ref[...] * (p_ref[0:N, 0:64] + p_ref[0:N, 64:128]
                        + 2.0 * g_ref[0:N]) + b_ref[...])
  out_ref[0:N, :] = dinv_ref[...] * jnp.dot(
      z, w2_ref[...], preferred_element_type=jnp.float32)
  out_ref[N:NRJ, :] = jnp.zeros((NRJ - N, w2_ref.shape[1]), jnp.float32)


def _final_body(p_ref, g_ref, dinv_ref, b_ref, out_ref):
  v = (dinv_ref[...] * (p_ref[0:N, 0:48] + p_ref[0:N, 48:96]
                        + 2.0 * g_ref[0:N]) + b_ref[...])
  out_ref[...] = v[:, 0:40]


def kernel(x, edge_index, W1, b1, W2, b2, W3, b3, W4, b4):
  i32 = jnp.int32
  f32 = jnp.float32
  src_pad = N + jnp.arange(EPAD - E, dtype=i32) % (NRJ - N)
  dst_pad = jnp.arange(EPAD - E, dtype=i32)
  src80 = edge_index[0].reshape(NW, 125, 80)
  dst80 = edge_index[1].reshape(NW, 125, 80)
  src128 = jnp.concatenate([edge_index[0], src_pad]).reshape(NW, 79, 128)
  dst128 = jnp.concatenate([edge_index[1], dst_pad]).reshape(NW, 79, 128)
  dst_deg = edge_index[1].reshape(NW, DEG_NCHUNK, DEG_CHUNK)

  zeros = {d: jnp.zeros((NRA, d), f32) for d in (128, 64, 48)}
  zeros16 = jnp.zeros((NRA, DEG_D), f32)
  ones16 = jnp.ones((DEG_CHUNK, DEG_D), f32)
  W4p = jnp.pad(W4, ((0, 0), (0, 8)))
  b4p = jnp.pad(b4, (0, 8))

  dp = _DEGREE(ones16, dst_deg, zeros16)
  dinv, g0 = _tc_call(
      _deg_scale_body,
      (jax.ShapeDtypeStruct((N, 1), f32),
       jax.ShapeDtypeStruct((N, 128), f32)),
      dp, x)

  s0 = _PROPAGATE[128](g0, src80, dst80, zeros[128])
  g1 = _tc_call(_layer_relu_mm_body, jax.ShapeDtypeStruct((N, 128), f32),
                s0, g0, dinv, W1, b1.reshape(1, -1), W2)

  s1 = _PROPAGATE[128](g1, src80, dst80, zeros[128])
  g2 = _tc_call(_layer_relu_ew_mm_body, jax.ShapeDtypeStruct((NRJ, 64), f32),
                s1, g1, dinv, b2.reshape(1, -1), W3)

  s2 = _PROPAGATE[64](g2, src128, dst128, zeros[64])
  g3 = _tc_call(_layer_lin_mm_body, jax.ShapeDtypeStruct((NRJ, 48), f32),
                s2, g2, dinv, b3.reshape(1, -1), W4p)

  s3 = _PROPAGATE[48](g3, src128, dst128, zeros[48])
  return _tc_call(_final_body, jax.ShapeDtypeStruct((N, 40), f32),
                  s3, g3, dinv, b4p.reshape(1, -1))


# speedup vs baseline: 1.5641x; 1.0135x over previous
import functools

import jax
import jax.numpy as jnp
from jax import lax
from jax.experimental import pallas as pl
from jax.experimental.pallas import tpu as pltpu
from jax.experimental.pallas import tpu_sc as plsc

N = 10000
E = 320000
NC = 2
NS = 16
NW = NC * NS
NRJ = N + 128
NRA = N + 8

PCHUNKS = {128: (80, 125, 3), 64: (128, 79, 4), 48: (128, 79, 4)}
EPAD = NW * 79 * 128

DEG_CHUNK = 80
DEG_NCHUNK = 125
DEG_D = 16
DEG_FIRE = 5

ROWS_PT = 624
ROWS_TAIL = NRA - NS * ROWS_PT


def _make_propagate(d):
  PCHUNK, PNCHUNK, NBUF = PCHUNKS[d]
  mesh = plsc.VectorSubcoreMesh(core_axis_name="c", subcore_axis_name="s")

  rows_bufs = [pltpu.VMEM((PCHUNK, d), jnp.float32) for _ in range(NBUF)]
  sems = [pltpu.SemaphoreType.DMA for _ in range(NBUF)]

  paired = d < 128
  out_shape = (NRA, 128) if paired else (NC, NRA, d)

  @functools.partial(
      pl.kernel,
      out_type=jax.ShapeDtypeStruct(out_shape, jnp.float32),
      mesh=mesh,
      scratch_types=[
          pltpu.VMEM((PNCHUNK, PCHUNK), jnp.int32),
          pltpu.VMEM((PNCHUNK, PCHUNK), jnp.int32),
          pltpu.VMEM_SHARED((NRA, d), jnp.float32),
      ] + rows_bufs + sems,
      compiler_params=pltpu.CompilerParams(use_tc_tiling_on_sc=False),
  )
  def propagate(g_hbm, src_hbm, dst_hbm, zeros_hbm, out_hbm,
                src_v, dst_v, acc, *bufs_sems):
    rows = bufs_sems[:NBUF]
    sem = bufs_sems[NBUF:]
    cid = lax.axis_index("c")
    sid = lax.axis_index("s")
    wid = cid * NS + sid
    pltpu.sync_copy(src_hbm.at[wid], src_v)
    pltpu.sync_copy(dst_hbm.at[wid], dst_v)
    row0 = sid * ROWS_PT
    pltpu.sync_copy(zeros_hbm.at[pl.ds(row0, ROWS_PT)],
                    acc.at[pl.ds(row0, ROWS_PT)])

    @pl.when(sid == 0)
    def _zero_tail():
      pltpu.sync_copy(zeros_hbm.at[pl.ds(NS * ROWS_PT, ROWS_TAIL)],
                      acc.at[pl.ds(NS * ROWS_PT, ROWS_TAIL)])

    plsc.subcore_barrier()

    for c in range(NBUF):
      pltpu.async_copy(g_hbm.at[src_v.at[c]], rows[c], sem[c])

    loop_end = (PNCHUNK // NBUF) * NBUF

    @pl.loop(0, loop_end, step=NBUF)
    def _group(j):
      for b in range(NBUF):
        pltpu.make_async_copy(g_hbm.at[src_v.at[j + b]], rows[b],
                              sem[b]).wait()
        pltpu.sync_copy(rows[b], acc.at[dst_v.at[j + b]], add=True)

        @pl.when(j + b + NBUF < PNCHUNK)
        def _next():
          pltpu.async_copy(g_hbm.at[src_v.at[j + b + NBUF]], rows[b], sem[b])

    for c in range(loop_end, PNCHUNK):
      b = c % NBUF
      pltpu.make_async_copy(g_hbm.at[src_v.at[c]], rows[b], sem[b]).wait()
      pltpu.sync_copy(rows[b], acc.at[dst_v.at[c]], add=True)

    plsc.subcore_barrier()
    if paired:
      pltpu.sync_copy(acc.at[pl.ds(row0, ROWS_PT)],
                      out_hbm.at[pl.ds(row0, ROWS_PT), pl.ds(cid * d, d)])

      @pl.when(sid == 0)
      def _copy_tail():
        pltpu.sync_copy(acc.at[pl.ds(NS * ROWS_PT, ROWS_TAIL)],
                        out_hbm.at[pl.ds(NS * ROWS_PT, ROWS_TAIL),
                                   pl.ds(cid * d, d)])
    else:
      pltpu.sync_copy(acc.at[pl.ds(row0, ROWS_PT)],
                      out_hbm.at[cid, pl.ds(row0, ROWS_PT)])

      @pl.when(sid == 0)
      def _copy_tail():
        pltpu.sync_copy(acc.at[pl.ds(NS * ROWS_PT, ROWS_TAIL)],
                        out_hbm.at[cid, pl.ds(NS * ROWS_PT, ROWS_TAIL)])

  return propagate


_PROPAGATE = {d: _make_propagate(d) for d in (128, 64, 48)}


def _make_degree():
  mesh = plsc.VectorSubcoreMesh(core_axis_name="c", subcore_axis_name="s")

  @functools.partial(
      pl.kernel,
      out_type=jax.ShapeDtypeStruct((NRA, 128), jnp.float32),
      mesh=mesh,
      scratch_types=[
          pltpu.VMEM((DEG_NCHUNK, DEG_CHUNK), jnp.int32),
          pltpu.VMEM((DEG_CHUNK, DEG_D), jnp.float32),
          pltpu.VMEM_SHARED((NRA, DEG_D), jnp.float32),
          pltpu.SemaphoreType.DMA,
      ],
      compiler_params=pltpu.CompilerParams(use_tc_tiling_on_sc=False),
  )
  def degree(ones_hbm, dst_hbm, zeros_hbm, out_hbm, dst_v, ones_v, acc, sem):
    cid = lax.axis_index("c")
    sid = lax.axis_index("s")
    wid = cid * NS + sid
    pltpu.sync_copy(dst_hbm.at[wid], dst_v)
    pltpu.sync_copy(ones_hbm, ones_v)
    row0 = sid * ROWS_PT
    pltpu.sync_copy(zeros_hbm.at[pl.ds(row0, ROWS_PT)],
                    acc.at[pl.ds(row0, ROWS_PT)])

    @pl.when(sid == 0)
    def _zero_tail():
      pltpu.sync_copy(zeros_hbm.at[pl.ds(NS * ROWS_PT, ROWS_TAIL)],
                      acc.at[pl.ds(NS * ROWS_PT, ROWS_TAIL)])

    plsc.subcore_barrier()

    @pl.loop(0, DEG_NCHUNK, step=DEG_FIRE)
    def _group(j):
      for k in range(DEG_FIRE):
        pltpu.async_copy(ones_v, acc.at[dst_v.at[j + k]], sem, add=True)
      for k in range(DEG_FIRE):
        pltpu.make_async_copy(ones_v, acc.at[dst_v.at[j + k]], sem).wait()

    plsc.subcore_barrier()
    pltpu.sync_copy(acc.at[pl.ds(row0, ROWS_PT)],
                    out_hbm.at[pl.ds(row0, ROWS_PT), pl.ds(cid * DEG_D, DEG_D)])

    @pl.when(sid == 0)
    def _copy_tail():
      pltpu.sync_copy(acc.at[pl.ds(NS * ROWS_PT, ROWS_TAIL)],
                      out_hbm.at[pl.ds(NS * ROWS_PT, ROWS_TAIL),
                                 pl.ds(cid * DEG_D, DEG_D)])

  return degree


_DEGREE = _make_degree()


def _tc_call(body, out_shape, *args):
  return pl.pallas_call(body, out_shape=out_shape)(*args)


def _deg_scale_body(dp_ref, x_ref, dinv_ref, g_ref):
  deg = dp_ref[0:N, 0:1] + dp_ref[0:N, DEG_D:DEG_D + 1] + 2.0
  dinv = lax.rsqrt(deg)
  dinv_ref[...] = dinv
  g_ref[...] = x_ref[...] * dinv


def _layer_relu_mm_body(p_ref, g_ref, dinv_ref, w_ref, b_ref, w2_ref, out_ref):
  s = dinv_ref[...] * (p_ref[0, 0:N] + p_ref[1, 0:N] + 2.0 * g_ref[...])
  z = jax.nn.relu(
      jnp.dot(s, w_ref[...], preferred_element_type=jnp.float32) + b_ref[...])
  out_ref[...] = dinv_ref[...] * jnp.dot(
      z, w2_ref[...], preferred_element_type=jnp.float32)


def _layer_relu_ew_mm_body(p_ref, g_ref, dinv_ref, b_ref, w2_ref, out_ref):
  z = jax.nn.relu(
      dinv_ref[...] * (p_ref[0, 0:N] + p_ref[1, 0:N] + 2.0 * g_ref[...])
      + b_ref[...])
  out_ref[0:N, :] = dinv_ref[...] * jnp.dot(
      z, w2_ref[...], preferred_element_type=jnp.float32)
  out_ref[N:NRJ, :] = jnp.zeros((NRJ - N, w2_ref.shape[1]), jnp.float32)


def _layer_lin_mm_body(p_ref, g_ref, dinv_ref, b_ref, w2_ref, out_ref):
  z = (dinv_ref[...] * (p_ref[0:N, 0:64] + p_ref[0:N, 64:128]
                        + 2.0 * g_ref[0:N]) + b_ref[...])
  out_ref[0:N, :] = dinv_ref[...] * jnp.dot(
      z, w2_ref[...], preferred_element_type=jnp.float32)
  out_ref[N:NRJ, :] = jnp.zeros((NRJ - N, w2_ref.shape[1]), jnp.float32)


def _final_body(p_ref, g_ref, dinv_ref, b_ref, out_ref):
  v = (dinv_ref[...] * (p_ref[0:N, 0:48] + p_ref[0:N, 48:96]
                        + 2.0 * g_ref[0:N]) + b_ref[...])
  out_ref[...] = v[:, 0:40]


def kernel(x, edge_index, W1, b1, W2, b2, W3, b3, W4, b4):
  i32 = jnp.int32
  f32 = jnp.float32
  src_pad = N + jnp.arange(EPAD - E, dtype=i32) % (NRJ - N)
  dst_pad = jnp.arange(EPAD - E, dtype=i32)
  src80 = edge_index[0].reshape(NW, 125, 80)
  dst80 = edge_index[1].reshape(NW, 125, 80)
  src128 = jnp.concatenate([edge_index[0], src_pad]).reshape(NW, 79, 128)
  dst128 = jnp.concatenate([edge_index[1], dst_pad]).reshape(NW, 79, 128)
  dst_deg = edge_index[1].reshape(NW, DEG_NCHUNK, DEG_CHUNK)

  zeros = {d: jnp.zeros((NRA, d), f32) for d in (128, 64, 48)}
  zeros16 = jnp.zeros((NRA, DEG_D), f32)
  ones16 = jnp.ones((DEG_CHUNK, DEG_D), f32)
  W4p = jnp.pad(W4, ((0, 0), (0, 8)))
  b4p = jnp.pad(b4, (0, 8))

  dp = _DEGREE(ones16, dst_deg, zeros16)
  dinv, g0 = _tc_call(
      _deg_scale_body,
      (jax.ShapeDtypeStruct((N, 1), f32),
       jax.ShapeDtypeStruct((N, 128), f32)),
      dp, x)

  s0 = _PROPAGATE[128](g0, src80, dst80, zeros[128])
  g1 = _tc_call(_layer_relu_mm_body, jax.ShapeDtypeStruct((N, 128), f32),
                s0, g0, dinv, W1, b1.reshape(1, -1), W2)

  s1 = _PROPAGATE[128](g1, src80, dst80, zeros[128])
  g2 = _tc_call(_layer_relu_ew_mm_body, jax.ShapeDtypeStruct((NRJ, 64), f32),
                s1, g1, dinv, b2.reshape(1, -1), W3)

  s2 = _PROPAGATE[64](g2, src128, dst128, zeros[64])
  g3 = _tc_call(_layer_lin_mm_body, jax.ShapeDtypeStruct((NRJ, 48), f32),
                s2, g2, dinv, b3.reshape(1, -1), W4p)

  s3 = _PROPAGATE[48](g3, src128, dst128, zeros[48])
  return _tc_call(_final_body, jax.ShapeDtypeStruct((N, 40), f32),
                  s3, g3, dinv, b4p.reshape(1, -1))
